# baseline (speedup 1.0000x reference)
import functools

import jax
import jax.numpy as jnp
from jax import lax
from jax.experimental import pallas as pl
from jax.experimental.pallas import tpu as pltpu
from jax.experimental.pallas import tpu_sc as plsc

_H = 16
_WROW = 24
_K = 80


def _emit_tables(h, beta, src_t_ref, dst_t_ref, seed_ref):
    r = h.shape[0]
    s2 = jnp.sum(h * h, axis=1, keepdims=True)
    n = jnp.maximum(jnp.sqrt(s2), 1e-12)
    hn = h / n
    s_self = jnp.sum(hn * hn, axis=1, keepdims=True)
    ex0 = jnp.exp(beta * s_self)
    z15 = jnp.zeros((r, _WROW - _H - 1), jnp.float32)
    src_t_ref[...] = jnp.concatenate([hn, n, z15], axis=1)
    dst_t_ref[...] = beta * hn
    seed_ref[...] = jnp.concatenate([0.5 * ex0 * h, 0.5 * ex0, z15], axis=1)


def _pre_body(x_ref, w1_ref, b1_ref, beta_ref, src_t_ref, dst_t_ref, seed_ref):
    h = jnp.dot(x_ref[...], w1_ref[...], preferred_element_type=jnp.float32)
    h = jnp.maximum(h + b1_ref[...], 0.0)
    _emit_tables(h, beta_ref[...], src_t_ref, dst_t_ref, seed_ref)


def _combine(p0, p1):
    s = p0 + p1
    return s[:, :_H] / s[:, _H:_H + 1]


def _mid_body(p0_ref, p1_ref, beta_ref, src_t_ref, dst_t_ref, seed_ref):
    h = _combine(p0_ref[...], p1_ref[...])
    _emit_tables(h, beta_ref[...], src_t_ref, dst_t_ref, seed_ref)


_RBLK = 1000


def _table_specs(n):
    out_specs = [
        pl.BlockSpec((_RBLK, _WROW), lambda i: (i, 0)),
        pl.BlockSpec((_RBLK, _H), lambda i: (i, 0)),
        pl.BlockSpec((_RBLK, _WROW), lambda i: (i, 0)),
    ]
    out_shape = [
        jax.ShapeDtypeStruct((n, _WROW), jnp.float32),
        jax.ShapeDtypeStruct((n, _H), jnp.float32),
        jax.ShapeDtypeStruct((n, _WROW), jnp.float32),
    ]
    return out_specs, out_shape


def _post_body(p0_ref, p1_ref, w2_ref, b2_ref, out_ref):
    h = _combine(p0_ref[...], p1_ref[...])
    logits = jnp.dot(h, w2_ref[...], preferred_element_type=jnp.float32)
    logits = logits + b2_ref[...]
    m = jnp.max(logits, axis=1, keepdims=True)
    e = jnp.exp(logits - m)
    lse = jnp.log(jnp.sum(e, axis=1, keepdims=True)) + m
    out_ref[...] = logits - lse


def _tc_pre(x, w1, b1, beta, d, n):
    out_specs, out_shape = _table_specs(n)
    return pl.pallas_call(
        _pre_body,
        grid=(n // _RBLK,),
        in_specs=[
            pl.BlockSpec((_RBLK, d), lambda i: (i, 0)),
            pl.BlockSpec((d, _H), lambda i: (0, 0)),
            pl.BlockSpec((1, _H), lambda i: (0, 0)),
            pl.BlockSpec((1, 1), lambda i: (0, 0)),
        ],
        out_specs=out_specs,
        out_shape=out_shape,
    )(x, w1, b1, beta)


def _tc_mid(p0, p1, beta, n):
    out_specs, out_shape = _table_specs(n)
    return pl.pallas_call(
        _mid_body,
        grid=(n // _RBLK,),
        in_specs=[
            pl.BlockSpec((_RBLK, _WROW), lambda i: (i, 0)),
            pl.BlockSpec((_RBLK, _WROW), lambda i: (i, 0)),
            pl.BlockSpec((1, 1), lambda i: (0, 0)),
        ],
        out_specs=out_specs,
        out_shape=out_shape,
    )(p0, p1, beta)


def _tc_post(p0, p1, w2, b2, n, c):
    return pl.pallas_call(
        _post_body,
        grid=(n // _RBLK,),
        in_specs=[
            pl.BlockSpec((_RBLK, _WROW), lambda i: (i, 0)),
            pl.BlockSpec((_RBLK, _WROW), lambda i: (i, 0)),
            pl.BlockSpec((_H, c), lambda i: (0, 0)),
            pl.BlockSpec((1, c), lambda i: (0, 0)),
        ],
        out_specs=pl.BlockSpec((_RBLK, c), lambda i: (i, 0)),
        out_shape=jax.ShapeDtypeStruct((n, c), jnp.float32),
    )(p0, p1, w2, b2)


def _make_sc_prop(n_pad, e):
    n_tiles = 32
    ept = e // n_tiles
    nblk = ept // _K
    rpt = n_pad // 16
    mesh = plsc.VectorSubcoreMesh(core_axis_name="c", subcore_axis_name="s")

    nbuf = 5
    look = 3

    def body(src_t, dst_t, seed, src_e, dst_e, out, idxs_all, idxd_all,
             s_rows, d_rows, o_rows, acc, *sems):
        sg = sems[:nbuf]
        ss = sems[nbuf:]
        c = lax.axis_index("c")
        s = lax.axis_index("s")
        wid = c * 16 + s

        zero16 = jnp.zeros((16,), jnp.float32)
        lanes0 = lax.iota(jnp.int32, 16)

        def zrow(g, carry):
            rows = g * 16 + lanes0
            for k in range(nbuf):
                for j in range(_H + 1, _WROW):
                    plsc.store_scatter(
                        o_rows[k], [rows, jnp.full((16,), j, jnp.int32)], zero16)
            return carry

        lax.fori_loop(0, _K // 16, zrow, 0)

        pltpu.sync_copy(src_e.at[wid], idxs_all)
        pltpu.sync_copy(dst_e.at[wid], idxd_all)
        r0 = s * rpt
        pltpu.sync_copy(seed.at[pl.ds(r0, rpt)], acc.at[pl.ds(r0, rpt)])
        plsc.subcore_barrier()

        lanes = lax.iota(jnp.int32, 16)

        def issue_gather(b, slot):
            pltpu.async_copy(src_t.at[idxs_all.at[b]], s_rows[slot], sg[slot])
            pltpu.async_copy(dst_t.at[idxd_all.at[b]], d_rows[slot], sg[slot])

        def wait_gather(b, slot):
            pltpu.make_async_copy(src_t.at[idxs_all.at[b]], s_rows[slot], sg[slot]).wait()
            pltpu.make_async_copy(dst_t.at[idxd_all.at[b]], d_rows[slot], sg[slot]).wait()

        def issue_scatter(b, slot):
            pltpu.async_copy(o_rows[slot], acc.at[idxd_all.at[b]], ss[slot], add=True)

        def wait_scatter(b, slot):
            pltpu.make_async_copy(o_rows[slot], acc.at[idxd_all.at[b]], ss[slot]).wait()

        for p in range(look):
            issue_gather(p, p)

        def superblock(sb, carry):
            for k in range(nbuf):
                b = sb * nbuf + k

                @pl.when(b + look < nblk)
                def _():
                    issue_gather(b + look, (k + look) % nbuf)

                wait_gather(b, k)

                @pl.when(sb >= 1)
                def _():
                    wait_scatter(b, k)

                for g in range(_K // 16):
                    rows = g * 16 + lanes
                    a = []
                    dot = jnp.zeros((16,), jnp.float32)
                    for j in range(_H):
                        col = jnp.full((16,), j, jnp.int32)
                        aj = plsc.load_gather(s_rows[k], [rows, col])
                        bj = plsc.load_gather(d_rows[k], [rows, col])
                        a.append(aj)
                        dot = dot + aj * bj
                    ex = jnp.exp(dot)
                    nsrc = plsc.load_gather(
                        s_rows[k], [rows, jnp.full((16,), _H, jnp.int32)])
                    scale = ex * nsrc
                    for j in range(_H):
                        col = jnp.full((16,), j, jnp.int32)
                        plsc.store_scatter(o_rows[k], [rows, col], a[j] * scale)
                    plsc.store_scatter(
                        o_rows[k], [rows, jnp.full((16,), _H, jnp.int32)], ex)

                issue_scatter(b, k)
            return carry

        lax.fori_loop(0, nblk // nbuf, superblock, 0)
        for k in range(nbuf):
            wait_scatter(nblk - nbuf + k, k)
        plsc.subcore_barrier()

        pltpu.sync_copy(acc.at[pl.ds(r0, rpt)], out.at[c, pl.ds(r0, rpt)])

    return functools.partial(
        pl.kernel,
        out_type=jax.ShapeDtypeStruct((2, n_pad, _WROW), jnp.float32),
        mesh=mesh,
        compiler_params=pltpu.CompilerParams(
            needs_layout_passes=False, use_tc_tiling_on_sc=False),
        scratch_types=[
            pltpu.VMEM((nblk, _K), jnp.int32),
            pltpu.VMEM((nblk, _K), jnp.int32),
            [pltpu.VMEM((_K, _WROW), jnp.float32) for _ in range(nbuf)],
            [pltpu.VMEM((_K, _H), jnp.float32) for _ in range(nbuf)],
            [pltpu.VMEM((_K, _WROW), jnp.float32) for _ in range(nbuf)],
            pltpu.VMEM_SHARED((n_pad, _WROW), jnp.float32),
        ] + [pltpu.SemaphoreType.DMA] * (2 * nbuf),
    )(body)


def kernel(x, edge_index, W1, b1, W2, b2, beta2):
    n, d = x.shape
    e = edge_index.shape[1]
    h = W1.shape[1]
    c = W2.shape[1]

    src = edge_index[0].reshape(32, -1, _K)
    dst = edge_index[1].reshape(32, -1, _K)
    b1r = b1.reshape(1, h)
    b2r = b2.reshape(1, c)
    one = jnp.ones((1, 1), jnp.float32)
    beta2r = beta2.reshape(1, 1).astype(jnp.float32)

    n_pad = ((n // 16 + 7) // 8 * 8) * 16
    sc_prop = _make_sc_prop(n_pad, e)

    zpad = jnp.zeros((n_pad - n, _WROW), jnp.float32)
    src_t, dst_t, seed = _tc_pre(x, W1, b1r, one, d, n)
    p = sc_prop(src_t, dst_t, jnp.concatenate([seed, zpad]), src, dst)
    src_t1, dst_t1, seed1 = _tc_mid(p[0, :n], p[1, :n], beta2r, n)
    p2 = sc_prop(src_t1, dst_t1, jnp.concatenate([seed1, zpad]), src, dst)
    return _tc_post(p2[0, :n], p2[1, :n], W2, b2r, n, c)


# speedup vs baseline: 1.2819x; 1.2819x over previous
import functools

import jax
import jax.numpy as jnp
from jax import lax
from jax.experimental import pallas as pl
from jax.experimental.pallas import tpu as pltpu
from jax.experimental.pallas import tpu_sc as plsc

_H = 16
_WROW = 24
_K = 80


def _emit_tables(h, beta, src_t_ref, dst_t_ref, seed_ref):
    r = h.shape[0]
    s2 = jnp.sum(h * h, axis=1, keepdims=True)
    n = jnp.maximum(jnp.sqrt(s2), 1e-12)
    hn = h / n
    s_self = jnp.sum(hn * hn, axis=1, keepdims=True)
    ex0 = jnp.exp(beta * s_self)
    z15 = jnp.zeros((r, _WROW - _H - 1), jnp.float32)
    src_t_ref[...] = jnp.concatenate([hn, n, z15], axis=1)
    dst_t_ref[...] = beta * hn
    seed_ref[...] = jnp.concatenate([0.5 * ex0 * h, 0.5 * ex0, z15], axis=1)


def _pre_body(x_ref, w1_ref, b1_ref, beta_ref, src_t_ref, dst_t_ref, seed_ref):
    h = jnp.dot(x_ref[...], w1_ref[...], preferred_element_type=jnp.float32)
    h = jnp.maximum(h + b1_ref[...], 0.0)
    _emit_tables(h, beta_ref[...], src_t_ref, dst_t_ref, seed_ref)


def _combine(p0, p1):
    s = p0 + p1
    return s[:, :_H] / s[:, _H:_H + 1]


def _mid_body(p0_ref, p1_ref, beta_ref, src_t_ref, dst_t_ref, seed_ref):
    h = _combine(p0_ref[...], p1_ref[...])
    _emit_tables(h, beta_ref[...], src_t_ref, dst_t_ref, seed_ref)


_RBLK = 1000


def _table_specs(n):
    out_specs = [
        pl.BlockSpec((_RBLK, _WROW), lambda i: (i, 0)),
        pl.BlockSpec((_RBLK, _H), lambda i: (i, 0)),
        pl.BlockSpec((_RBLK, _WROW), lambda i: (i, 0)),
    ]
    out_shape = [
        jax.ShapeDtypeStruct((n, _WROW), jnp.float32),
        jax.ShapeDtypeStruct((n, _H), jnp.float32),
        jax.ShapeDtypeStruct((n, _WROW), jnp.float32),
    ]
    return out_specs, out_shape


def _post_body(p0_ref, p1_ref, w2_ref, b2_ref, out_ref):
    h = _combine(p0_ref[...], p1_ref[...])
    logits = jnp.dot(h, w2_ref[...], preferred_element_type=jnp.float32)
    logits = logits + b2_ref[...]
    m = jnp.max(logits, axis=1, keepdims=True)
    e = jnp.exp(logits - m)
    lse = jnp.log(jnp.sum(e, axis=1, keepdims=True)) + m
    out_ref[...] = logits - lse


def _tc_pre(x, w1, b1, beta, d, n):
    out_specs, out_shape = _table_specs(n)
    return pl.pallas_call(
        _pre_body,
        grid=(n // _RBLK,),
        in_specs=[
            pl.BlockSpec((_RBLK, d), lambda i: (i, 0)),
            pl.BlockSpec((d, _H), lambda i: (0, 0)),
            pl.BlockSpec((1, _H), lambda i: (0, 0)),
            pl.BlockSpec((1, 1), lambda i: (0, 0)),
        ],
        out_specs=out_specs,
        out_shape=out_shape,
    )(x, w1, b1, beta)


def _tc_mid(p0, p1, beta, n):
    out_specs, out_shape = _table_specs(n)
    return pl.pallas_call(
        _mid_body,
        grid=(n // _RBLK,),
        in_specs=[
            pl.BlockSpec((_RBLK, _WROW), lambda i: (i, 0)),
            pl.BlockSpec((_RBLK, _WROW), lambda i: (i, 0)),
            pl.BlockSpec((1, 1), lambda i: (0, 0)),
        ],
        out_specs=out_specs,
        out_shape=out_shape,
    )(p0, p1, beta)


def _tc_post(p0, p1, w2, b2, n, c):
    return pl.pallas_call(
        _post_body,
        grid=(n // _RBLK,),
        in_specs=[
            pl.BlockSpec((_RBLK, _WROW), lambda i: (i, 0)),
            pl.BlockSpec((_RBLK, _WROW), lambda i: (i, 0)),
            pl.BlockSpec((_H, c), lambda i: (0, 0)),
            pl.BlockSpec((1, c), lambda i: (0, 0)),
        ],
        out_specs=pl.BlockSpec((_RBLK, c), lambda i: (i, 0)),
        out_shape=jax.ShapeDtypeStruct((n, c), jnp.float32),
    )(p0, p1, w2, b2)


def _make_sc_prop(n_pad, e):
    n_tiles = 32
    ept = e // n_tiles
    nblk = ept // _K
    rpt = n_pad // 16
    mesh = plsc.VectorSubcoreMesh(core_axis_name="c", subcore_axis_name="s")

    nbuf = 5
    look = 3

    def body(src_t, dst_t, seed, src_e, dst_e, out, idxs_all, idxd_all,
             s_rows, d_rows, o_rows, acc, *sems):
        sg = sems[:nbuf]
        ss = sems[nbuf:]
        c = lax.axis_index("c")
        s = lax.axis_index("s")
        wid = c * 16 + s

        zero16 = jnp.zeros((16,), jnp.float32)
        lanes0 = lax.iota(jnp.int32, 16)

        def zrow(g, carry):
            rows = g * 16 + lanes0
            for k in range(nbuf):
                for j in range(_H + 1, _WROW):
                    plsc.store_scatter(
                        o_rows[k], [rows, jnp.full((16,), j, jnp.int32)], zero16)
            return carry

        lax.fori_loop(0, _K // 16, zrow, 0)

        pltpu.sync_copy(src_e.at[wid], idxs_all)
        pltpu.sync_copy(dst_e.at[wid], idxd_all)
        r0 = s * rpt
        pltpu.sync_copy(seed.at[pl.ds(r0, rpt)], acc.at[pl.ds(r0, rpt)])
        plsc.subcore_barrier()

        lanes = lax.iota(jnp.int32, 16)

        def issue_gather(b, slot):
            pltpu.async_copy(src_t.at[idxs_all.at[b]], s_rows[slot], sg[slot])
            pltpu.async_copy(dst_t.at[idxd_all.at[b]], d_rows[slot], sg[slot])

        def wait_gather(b, slot):
            pltpu.make_async_copy(src_t.at[idxs_all.at[b]], s_rows[slot], sg[slot]).wait()
            pltpu.make_async_copy(dst_t.at[idxd_all.at[b]], d_rows[slot], sg[slot]).wait()

        def issue_scatter(b, slot):
            pltpu.async_copy(o_rows[slot], acc.at[idxd_all.at[b]], ss[slot], add=True)

        def wait_scatter(b, slot):
            pltpu.make_async_copy(o_rows[slot], acc.at[idxd_all.at[b]], ss[slot]).wait()

        for p in range(look):
            issue_gather(p, p)

        def superblock(sb, carry):
            for k in range(nbuf):
                b = sb * nbuf + k

                @pl.when(b + look < nblk)
                def _():
                    issue_gather(b + look, (k + look) % nbuf)

                wait_gather(b, k)

                @pl.when(sb >= 1)
                def _():
                    wait_scatter(b, k)

                def group(g, carry2):
                    rows = g * 16 + lanes
                    a = []
                    dot = jnp.zeros((16,), jnp.float32)
                    for j in range(_H):
                        col = jnp.full((16,), j, jnp.int32)
                        aj = plsc.load_gather(s_rows[k], [rows, col])
                        bj = plsc.load_gather(d_rows[k], [rows, col])
                        a.append(aj)
                        dot = dot + aj * bj
                    ex = jnp.exp(dot)
                    nsrc = plsc.load_gather(
                        s_rows[k], [rows, jnp.full((16,), _H, jnp.int32)])
                    scale = ex * nsrc
                    for j in range(_H):
                        col = jnp.full((16,), j, jnp.int32)
                        plsc.store_scatter(o_rows[k], [rows, col], a[j] * scale)
                    plsc.store_scatter(
                        o_rows[k], [rows, jnp.full((16,), _H, jnp.int32)], ex)
                    return carry2

                lax.fori_loop(0, _K // 16, group, 0)
                issue_scatter(b, k)
            return carry

        lax.fori_loop(0, nblk // nbuf, superblock, 0)
        for k in range(nbuf):
            wait_scatter(nblk - nbuf + k, k)
        plsc.subcore_barrier()

        pltpu.sync_copy(acc.at[pl.ds(r0, rpt)], out.at[c, pl.ds(r0, rpt)])

    return functools.partial(
        pl.kernel,
        out_type=jax.ShapeDtypeStruct((2, n_pad, _WROW), jnp.float32),
        mesh=mesh,
        compiler_params=pltpu.CompilerParams(
            needs_layout_passes=False, use_tc_tiling_on_sc=False),
        scratch_types=[
            pltpu.VMEM((nblk, _K), jnp.int32),
            pltpu.VMEM((nblk, _K), jnp.int32),
            [pltpu.VMEM((_K, _WROW), jnp.float32) for _ in range(nbuf)],
            [pltpu.VMEM((_K, _H), jnp.float32) for _ in range(nbuf)],
            [pltpu.VMEM((_K, _WROW), jnp.float32) for _ in range(nbuf)],
            pltpu.VMEM_SHARED((n_pad, _WROW), jnp.float32),
        ] + [pltpu.SemaphoreType.DMA] * (2 * nbuf),
    )(body)


def kernel(x, edge_index, W1, b1, W2, b2, beta2):
    n, d = x.shape
    e = edge_index.shape[1]
    h = W1.shape[1]
    c = W2.shape[1]

    src = edge_index[0].reshape(32, -1, _K)
    dst = edge_index[1].reshape(32, -1, _K)
    b1r = b1.reshape(1, h)
    b2r = b2.reshape(1, c)
    one = jnp.ones((1, 1), jnp.float32)
    beta2r = beta2.reshape(1, 1).astype(jnp.float32)

    n_pad = ((n // 16 + 7) // 8 * 8) * 16
    sc_prop = _make_sc_prop(n_pad, e)

    zpad = jnp.zeros((n_pad - n, _WROW), jnp.float32)
    src_t, dst_t, seed = _tc_pre(x, W1, b1r, one, d, n)
    p = sc_prop(src_t, dst_t, jnp.concatenate([seed, zpad]), src, dst)
    src_t1, dst_t1, seed1 = _tc_mid(p[0, :n], p[1, :n], beta2r, n)
    p2 = sc_prop(src_t1, dst_t1, jnp.concatenate([seed1, zpad]), src, dst)
    return _tc_post(p2[0, :n], p2[1, :n], W2, b2r, n, c)
